# baseline (speedup 1.0000x reference)
import functools

import jax
import jax.numpy as jnp
from jax import lax
from jax.experimental import pallas as pl
from jax.experimental.pallas import tpu as pltpu
from jax.experimental.pallas import tpu_sc as plsc

_DIM = 16
_BATCH = 16384
_N_SCORE = 32
_NUM_USERS = 1000000
_NUM_ITEMS = 100000

_NC, _NS = 2, 16
_NW = _NC * _NS
_B_W = _BATCH // _NW
_CHUNK = 128
_N_CHUNK = _B_W // _CHUNK

_BI = 12800

_SC_PARAMS = pltpu.CompilerParams(
    use_tc_tiling_on_sc=False, needs_layout_passes=False)


def _tc_split_dims(xt, blk, dep=None):
    n = xt.shape[1]

    def body(x_ref, *refs):
        out_refs = refs[1:] if dep is not None else refs
        for d in range(_DIM):
            out_refs[d][...] = x_ref[d, :]

    in_specs = [pl.BlockSpec((_DIM, blk), lambda i: (0, i))]
    args = [xt]
    if dep is not None:
        in_specs.append(pl.BlockSpec(memory_space=pl.ANY))
        args.append(dep)

    return pl.pallas_call(
        body,
        grid=(pl.cdiv(n, blk),),
        in_specs=in_specs,
        out_specs=[pl.BlockSpec((blk,), lambda i: (i,))] * _DIM,
        out_shape=[jax.ShapeDtypeStruct((n,), jnp.float32)] * _DIM,
    )(*args)


def _sc_stage_items(item_dims, pos_ids, neg_ids):
    mesh = plsc.VectorSubcoreMesh(core_axis_name="c", subcore_axis_name="s")

    @functools.partial(
        pl.kernel,
        mesh=mesh,
        compiler_params=_SC_PARAMS,
        out_type=[
            jax.ShapeDtypeStruct((_NW, _DIM, _B_W), jnp.float32),
            jax.ShapeDtypeStruct((_NW, _DIM, _B_W), jnp.float32),
        ],
        scratch_types=[
            pltpu.VMEM((_N_CHUNK, _CHUNK), jnp.int32),
            pltpu.VMEM((_N_CHUNK, _CHUNK), jnp.int32),
            pltpu.VMEM((_DIM, _B_W), jnp.float32),
            pltpu.VMEM((_DIM, _B_W), jnp.float32),
            pltpu.SemaphoreType.DMA,
        ],
    )
    def k(*refs):
        item_hbm = refs[:_DIM]
        (pid_hbm, nid_hbm, p_out, n_out,
         pid_v, nid_v, p_v, n_v, sem) = refs[_DIM:]
        wid = lax.axis_index("s") * _NC + lax.axis_index("c")
        base = wid * _B_W

        for c in range(_N_CHUNK):
            off = base + c * _CHUNK
            pltpu.sync_copy(pid_hbm.at[pl.ds(off, _CHUNK)], pid_v.at[c])
            pltpu.sync_copy(nid_hbm.at[pl.ds(off, _CHUNK)], nid_v.at[c])

        copies = []
        for d in range(_DIM):
            for c in range(_N_CHUNK):
                dst = pl.ds(c * _CHUNK, _CHUNK)
                copies.append(pltpu.async_copy(
                    item_hbm[d].at[pid_v.at[c]], p_v.at[d].at[dst], sem))
                copies.append(pltpu.async_copy(
                    item_hbm[d].at[nid_v.at[c]], n_v.at[d].at[dst], sem))
        for cp in copies:
            cp.wait()

        pltpu.sync_copy(p_v, p_out.at[wid])
        pltpu.sync_copy(n_v, n_out.at[wid])

    return k(*item_dims, pos_ids, neg_ids)


def _sc_score_users(user_dims, score_ids):
    mesh = plsc.VectorSubcoreMesh(core_axis_name="c", subcore_axis_name="s")

    @functools.partial(
        pl.kernel,
        mesh=mesh,
        compiler_params=_SC_PARAMS,
        out_type=jax.ShapeDtypeStruct((_DIM, _N_SCORE), jnp.float32),
        scratch_types=[
            pltpu.VMEM((_N_SCORE,), jnp.int32),
            pltpu.VMEM((_DIM, _N_SCORE), jnp.float32),
            pltpu.SemaphoreType.DMA,
        ],
    )
    def k(*refs):
        user_hbm = refs[:_DIM]
        sid_hbm, su_hbm, sid_v, su_v, sem = refs[_DIM:]
        wid = lax.axis_index("s") * _NC + lax.axis_index("c")

        @pl.when(wid == 0)
        def _():
            pltpu.sync_copy(sid_hbm, sid_v)
            copies = [
                pltpu.async_copy(user_hbm[d].at[sid_v], su_v.at[d], sem)
                for d in range(_DIM)
            ]
            for cp in copies:
                cp.wait()
            pltpu.sync_copy(su_v, su_hbm)

    return k(*user_dims, score_ids)


def _sc_dist_final(user_dims, p_stage, n_stage, user_ids):
    mesh = plsc.VectorSubcoreMesh(core_axis_name="c", subcore_axis_name="s")

    @functools.partial(
        pl.kernel,
        mesh=mesh,
        compiler_params=_SC_PARAMS,
        out_type=[
            jax.ShapeDtypeStruct((_BATCH,), jnp.float32),
            jax.ShapeDtypeStruct((_BATCH,), jnp.float32),
        ],
        scratch_types=[
            pltpu.VMEM((_N_CHUNK, _CHUNK), jnp.int32),
            pltpu.VMEM((_DIM, _B_W), jnp.float32),
            pltpu.VMEM((_DIM, _B_W), jnp.float32),
            pltpu.VMEM((_DIM, _B_W), jnp.float32),
            pltpu.VMEM((_B_W,), jnp.float32),
            pltpu.VMEM((_B_W,), jnp.float32),
            pltpu.SemaphoreType.DMA,
        ],
    )
    def k(*refs):
        user_hbm = refs[:_DIM]
        (p_hbm, n_hbm, uid_hbm, pos_hbm, neg_hbm,
         uid_v, u_v, p_v, n_v, pos_v, neg_v, sem) = refs[_DIM:]
        wid = lax.axis_index("s") * _NC + lax.axis_index("c")
        base = wid * _B_W

        for c in range(_N_CHUNK):
            off = base + c * _CHUNK
            pltpu.sync_copy(uid_hbm.at[pl.ds(off, _CHUNK)], uid_v.at[c])
        pltpu.sync_copy(p_hbm.at[wid], p_v)
        pltpu.sync_copy(n_hbm.at[wid], n_v)

        copies = []
        for d in range(_DIM):
            for c in range(_N_CHUNK):
                dst = pl.ds(c * _CHUNK, _CHUNK)
                copies.append(pltpu.async_copy(
                    user_hbm[d].at[uid_v.at[c]], u_v.at[d].at[dst], sem))
        for cp in copies:
            cp.wait()

        def body(g, carry):
            sl = pl.ds(g * 16, 16)
            accp = jnp.zeros((16,), jnp.float32)
            accn = jnp.zeros((16,), jnp.float32)
            for d in range(_DIM):
                u = u_v[d, sl]
                dp = u - p_v[d, sl]
                dn = u - n_v[d, sl]
                accp = accp + dp * dp
                accn = accn + dn * dn
            pos_v[sl] = accp
            neg_v[sl] = accn
            return carry

        lax.fori_loop(0, _B_W // 16, body, 0, unroll=2)

        pltpu.sync_copy(pos_v, pos_hbm.at[pl.ds(base, _B_W)])
        pltpu.sync_copy(neg_v, neg_hbm.at[pl.ds(base, _B_W)])

    return k(*user_dims, p_stage, n_stage, user_ids)


def _tc_scores(su_t, item_t):
    def body(su_ref, it_ref, out_ref):
        sut = su_ref[...]
        itb = it_ref[...]
        dots = lax.dot_general(sut, itb, (((0,), (0,)), ((), ())),
                               preferred_element_type=jnp.float32)
        su2 = jnp.sum(sut * sut, axis=0)
        it2 = jnp.sum(itb * itb, axis=0)
        out_ref[...] = 2.0 * dots - su2[:, None] - it2[None, :]

    return pl.pallas_call(
        body,
        grid=(pl.cdiv(_NUM_ITEMS, _BI),),
        in_specs=[
            pl.BlockSpec((_DIM, _N_SCORE), lambda i: (0, 0)),
            pl.BlockSpec((_DIM, _BI), lambda i: (0, i)),
        ],
        out_specs=pl.BlockSpec((_N_SCORE, _BI), lambda i: (0, i)),
        out_shape=jax.ShapeDtypeStruct((_N_SCORE, _NUM_ITEMS), jnp.float32),
    )(su_t, item_t)


def kernel(user_embeddings, item_embeddings, user_ids, pos_item_ids,
           neg_item_ids, score_user_ids):
    user_t = user_embeddings.T
    item_t = item_embeddings.T
    item_dims = _tc_split_dims(item_t, 20480)
    p_stage, n_stage = _sc_stage_items(item_dims, pos_item_ids, neg_item_ids)
    user_dims = _tc_split_dims(user_t, 131072, dep=item_dims[0])
    su_t = _sc_score_users(user_dims, score_user_ids)
    pos_d, neg_d = _sc_dist_final(user_dims, p_stage, n_stage, user_ids)
    scores = _tc_scores(su_t, item_t)
    return (pos_d, neg_d, scores)


# speedup vs baseline: 1.0025x; 1.0025x over previous
import functools

import jax
import jax.numpy as jnp
from jax import lax
from jax.experimental import pallas as pl
from jax.experimental.pallas import tpu as pltpu
from jax.experimental.pallas import tpu_sc as plsc

_DIM = 16
_BATCH = 16384
_N_SCORE = 32
_NUM_USERS = 1000000
_NUM_ITEMS = 100000

_NC, _NS = 2, 16
_NW = _NC * _NS
_B_W = _BATCH // _NW
_CHUNK = 128
_N_CHUNK = _B_W // _CHUNK

_BI = 25600

_SC_PARAMS = pltpu.CompilerParams(
    use_tc_tiling_on_sc=False, needs_layout_passes=False)


def _tc_split_dims(xt, blk, dep=None):
    n = xt.shape[1]

    def body(x_ref, *refs):
        out_refs = refs[1:] if dep is not None else refs
        for d in range(_DIM):
            out_refs[d][...] = x_ref[d, :]

    in_specs = [pl.BlockSpec((_DIM, blk), lambda i: (0, i))]
    args = [xt]
    if dep is not None:
        in_specs.append(pl.BlockSpec(memory_space=pl.ANY))
        args.append(dep)

    return pl.pallas_call(
        body,
        grid=(pl.cdiv(n, blk),),
        in_specs=in_specs,
        out_specs=[pl.BlockSpec((blk,), lambda i: (i,))] * _DIM,
        out_shape=[jax.ShapeDtypeStruct((n,), jnp.float32)] * _DIM,
    )(*args)


def _sc_stage_items(item_dims, pos_ids, neg_ids):
    mesh = plsc.VectorSubcoreMesh(core_axis_name="c", subcore_axis_name="s")

    @functools.partial(
        pl.kernel,
        mesh=mesh,
        compiler_params=_SC_PARAMS,
        out_type=[
            jax.ShapeDtypeStruct((_NW, _DIM, _B_W), jnp.float32),
            jax.ShapeDtypeStruct((_NW, _DIM, _B_W), jnp.float32),
        ],
        scratch_types=[
            pltpu.VMEM((_N_CHUNK, _CHUNK), jnp.int32),
            pltpu.VMEM((_N_CHUNK, _CHUNK), jnp.int32),
            pltpu.VMEM((_DIM, _B_W), jnp.float32),
            pltpu.VMEM((_DIM, _B_W), jnp.float32),
            pltpu.SemaphoreType.DMA,
        ],
    )
    def k(*refs):
        item_hbm = refs[:_DIM]
        (pid_hbm, nid_hbm, p_out, n_out,
         pid_v, nid_v, p_v, n_v, sem) = refs[_DIM:]
        wid = lax.axis_index("s") * _NC + lax.axis_index("c")
        base = wid * _B_W

        for c in range(_N_CHUNK):
            off = base + c * _CHUNK
            pltpu.sync_copy(pid_hbm.at[pl.ds(off, _CHUNK)], pid_v.at[c])
            pltpu.sync_copy(nid_hbm.at[pl.ds(off, _CHUNK)], nid_v.at[c])

        copies = []
        for d in range(_DIM):
            for c in range(_N_CHUNK):
                dst = pl.ds(c * _CHUNK, _CHUNK)
                copies.append(pltpu.async_copy(
                    item_hbm[d].at[pid_v.at[c]], p_v.at[d].at[dst], sem))
                copies.append(pltpu.async_copy(
                    item_hbm[d].at[nid_v.at[c]], n_v.at[d].at[dst], sem))
        for cp in copies:
            cp.wait()

        pltpu.sync_copy(p_v, p_out.at[wid])
        pltpu.sync_copy(n_v, n_out.at[wid])

    return k(*item_dims, pos_ids, neg_ids)


def _sc_score_users(user_dims, score_ids):
    mesh = plsc.VectorSubcoreMesh(core_axis_name="c", subcore_axis_name="s")

    @functools.partial(
        pl.kernel,
        mesh=mesh,
        compiler_params=_SC_PARAMS,
        out_type=jax.ShapeDtypeStruct((_DIM, _N_SCORE), jnp.float32),
        scratch_types=[
            pltpu.VMEM((_N_SCORE,), jnp.int32),
            pltpu.VMEM((_DIM, _N_SCORE), jnp.float32),
            pltpu.SemaphoreType.DMA,
        ],
    )
    def k(*refs):
        user_hbm = refs[:_DIM]
        sid_hbm, su_hbm, sid_v, su_v, sem = refs[_DIM:]
        wid = lax.axis_index("s") * _NC + lax.axis_index("c")

        @pl.when(wid == 0)
        def _():
            pltpu.sync_copy(sid_hbm, sid_v)
            copies = [
                pltpu.async_copy(user_hbm[d].at[sid_v], su_v.at[d], sem)
                for d in range(_DIM)
            ]
            for cp in copies:
                cp.wait()
            pltpu.sync_copy(su_v, su_hbm)

    return k(*user_dims, score_ids)


def _sc_dist_final(user_dims, p_stage, n_stage, user_ids):
    mesh = plsc.VectorSubcoreMesh(core_axis_name="c", subcore_axis_name="s")

    @functools.partial(
        pl.kernel,
        mesh=mesh,
        compiler_params=_SC_PARAMS,
        out_type=[
            jax.ShapeDtypeStruct((_BATCH,), jnp.float32),
            jax.ShapeDtypeStruct((_BATCH,), jnp.float32),
        ],
        scratch_types=[
            pltpu.VMEM((_N_CHUNK, _CHUNK), jnp.int32),
            pltpu.VMEM((_DIM, _B_W), jnp.float32),
            pltpu.VMEM((_DIM, _B_W), jnp.float32),
            pltpu.VMEM((_DIM, _B_W), jnp.float32),
            pltpu.VMEM((_B_W,), jnp.float32),
            pltpu.VMEM((_B_W,), jnp.float32),
            pltpu.SemaphoreType.DMA,
        ],
    )
    def k(*refs):
        user_hbm = refs[:_DIM]
        (p_hbm, n_hbm, uid_hbm, pos_hbm, neg_hbm,
         uid_v, u_v, p_v, n_v, pos_v, neg_v, sem) = refs[_DIM:]
        wid = lax.axis_index("s") * _NC + lax.axis_index("c")
        base = wid * _B_W

        for c in range(_N_CHUNK):
            off = base + c * _CHUNK
            pltpu.sync_copy(uid_hbm.at[pl.ds(off, _CHUNK)], uid_v.at[c])
        pltpu.sync_copy(p_hbm.at[wid], p_v)
        pltpu.sync_copy(n_hbm.at[wid], n_v)

        copies = []
        for d in range(_DIM):
            for c in range(_N_CHUNK):
                dst = pl.ds(c * _CHUNK, _CHUNK)
                copies.append(pltpu.async_copy(
                    user_hbm[d].at[uid_v.at[c]], u_v.at[d].at[dst], sem))
        for cp in copies:
            cp.wait()

        def body(g, carry):
            sl = pl.ds(g * 16, 16)
            accp = jnp.zeros((16,), jnp.float32)
            accn = jnp.zeros((16,), jnp.float32)
            for d in range(_DIM):
                u = u_v[d, sl]
                dp = u - p_v[d, sl]
                dn = u - n_v[d, sl]
                accp = accp + dp * dp
                accn = accn + dn * dn
            pos_v[sl] = accp
            neg_v[sl] = accn
            return carry

        lax.fori_loop(0, _B_W // 16, body, 0, unroll=2)

        pltpu.sync_copy(pos_v, pos_hbm.at[pl.ds(base, _B_W)])
        pltpu.sync_copy(neg_v, neg_hbm.at[pl.ds(base, _B_W)])

    return k(*user_dims, p_stage, n_stage, user_ids)


def _tc_scores(su_t, item_t):
    def body(su_ref, it_ref, out_ref):
        sut = su_ref[...]
        itb = it_ref[...]
        dots = lax.dot_general(sut, itb, (((0,), (0,)), ((), ())),
                               preferred_element_type=jnp.float32)
        su2 = jnp.sum(sut * sut, axis=0)
        it2 = jnp.sum(itb * itb, axis=0)
        out_ref[...] = 2.0 * dots - su2[:, None] - it2[None, :]

    return pl.pallas_call(
        body,
        grid=(pl.cdiv(_NUM_ITEMS, _BI),),
        in_specs=[
            pl.BlockSpec((_DIM, _N_SCORE), lambda i: (0, 0)),
            pl.BlockSpec((_DIM, _BI), lambda i: (0, i)),
        ],
        out_specs=pl.BlockSpec((_N_SCORE, _BI), lambda i: (0, i)),
        out_shape=jax.ShapeDtypeStruct((_N_SCORE, _NUM_ITEMS), jnp.float32),
    )(su_t, item_t)


def kernel(user_embeddings, item_embeddings, user_ids, pos_item_ids,
           neg_item_ids, score_user_ids):
    user_t = user_embeddings.T
    item_t = item_embeddings.T
    item_dims = _tc_split_dims(item_t, 20480)
    p_stage, n_stage = _sc_stage_items(item_dims, pos_item_ids, neg_item_ids)
    user_dims = _tc_split_dims(user_t, 131072, dep=item_dims[0])
    su_t = _sc_score_users(user_dims, score_user_ids)
    pos_d, neg_d = _sc_dist_final(user_dims, p_stage, n_stage, user_ids)
    scores = _tc_scores(su_t, item_t)
    return (pos_d, neg_d, scores)


# speedup vs baseline: 1.0638x; 1.0612x over previous
import functools

import jax
import jax.numpy as jnp
from jax import lax
from jax.experimental import pallas as pl
from jax.experimental.pallas import tpu as pltpu
from jax.experimental.pallas import tpu_sc as plsc

_DIM = 16
_BATCH = 16384
_N_SCORE = 32
_NUM_USERS = 1000000
_NUM_ITEMS = 100000

_NC, _NS = 2, 16
_NW = _NC * _NS
_B_W = _BATCH // _NW
_CHUNK = 128
_N_CHUNK = _B_W // _CHUNK

_BI = 25600

_SC_PARAMS = pltpu.CompilerParams(
    use_tc_tiling_on_sc=False, needs_layout_passes=False)


def _tc_split_dims(xt, blk, dep=None):
    n = xt.shape[1]

    def body(x_ref, *refs):
        out_refs = refs[1:] if dep is not None else refs
        for d in range(_DIM):
            out_refs[d][...] = x_ref[d, :]

    in_specs = [pl.BlockSpec((_DIM, blk), lambda i: (0, i))]
    args = [xt]
    if dep is not None:
        in_specs.append(pl.BlockSpec(memory_space=pl.ANY))
        args.append(dep)

    return pl.pallas_call(
        body,
        grid=(pl.cdiv(n, blk),),
        in_specs=in_specs,
        out_specs=[pl.BlockSpec((blk,), lambda i: (i,))] * _DIM,
        out_shape=[jax.ShapeDtypeStruct((n,), jnp.float32)] * _DIM,
    )(*args)


def _sc_stage_items(item_dims, pos_ids, neg_ids):
    mesh = plsc.VectorSubcoreMesh(core_axis_name="c", subcore_axis_name="s")

    @functools.partial(
        pl.kernel,
        mesh=mesh,
        compiler_params=_SC_PARAMS,
        out_type=[
            jax.ShapeDtypeStruct((_NW, _DIM, _B_W), jnp.float32),
            jax.ShapeDtypeStruct((_NW, _DIM, _B_W), jnp.float32),
        ],
        scratch_types=[
            pltpu.VMEM((_N_CHUNK, _CHUNK), jnp.int32),
            pltpu.VMEM((_N_CHUNK, _CHUNK), jnp.int32),
            pltpu.VMEM((_DIM, _B_W), jnp.float32),
            pltpu.VMEM((_DIM, _B_W), jnp.float32),
            pltpu.SemaphoreType.DMA,
        ],
    )
    def k(*refs):
        item_hbm = refs[:_DIM]
        (pid_hbm, nid_hbm, p_out, n_out,
         pid_v, nid_v, p_v, n_v, sem) = refs[_DIM:]
        wid = lax.axis_index("s") * _NC + lax.axis_index("c")
        base = wid * _B_W

        for c in range(_N_CHUNK):
            off = base + c * _CHUNK
            pltpu.sync_copy(pid_hbm.at[pl.ds(off, _CHUNK)], pid_v.at[c])
            pltpu.sync_copy(nid_hbm.at[pl.ds(off, _CHUNK)], nid_v.at[c])

        copies = []
        for d in range(_DIM):
            for c in range(_N_CHUNK):
                dst = pl.ds(c * _CHUNK, _CHUNK)
                copies.append(pltpu.async_copy(
                    item_hbm[d].at[pid_v.at[c]], p_v.at[d].at[dst], sem))
                copies.append(pltpu.async_copy(
                    item_hbm[d].at[nid_v.at[c]], n_v.at[d].at[dst], sem))
        for cp in copies:
            cp.wait()

        pltpu.sync_copy(p_v, p_out.at[wid])
        pltpu.sync_copy(n_v, n_out.at[wid])

    return k(*item_dims, pos_ids, neg_ids)


def _tc_split_user(xt, blk, score_ids, dep):
    n = xt.shape[1]
    grid = pl.cdiv(n, blk)

    def body(x_ref, sid_ref, dep_ref, *refs):
        out_refs = refs[:_DIM]
        su_ref = refs[_DIM]
        su_sc, w_sc, sem = refs[_DIM + 1:]
        i = pl.program_id(0)
        for d in range(_DIM):
            out_refs[d][...] = x_ref[d, :]

        lane128 = lax.broadcasted_iota(jnp.int32, (_DIM, 128), 1)
        col_iota = lax.broadcasted_iota(jnp.int32, (_DIM, _N_SCORE), 1)
        for s in range(_N_SCORE):
            sid = sid_ref[s]
            blk_id = sid // blk
            in_blk = sid - blk_id * blk
            off128 = pl.multiple_of(in_blk // 128 * 128, 128)
            lane = in_blk - off128

            @pl.when(i == blk_id)
            def _(off128=off128, lane=lane, s=s):
                pltpu.make_async_copy(
                    x_ref.at[:, pl.ds(off128, 128)], w_sc, sem).start()
                pltpu.make_async_copy(
                    x_ref.at[:, pl.ds(off128, 128)], w_sc, sem).wait()
                col = jnp.sum(jnp.where(lane128 == lane, w_sc[...], 0.0),
                              axis=1)
                su_sc[...] = jnp.where(col_iota == s, col[:, None], su_sc[...])

        @pl.when(i == grid - 1)
        def _():
            su_ref[...] = su_sc[...]

    return pl.pallas_call(
        body,
        grid=(grid,),
        in_specs=[
            pl.BlockSpec((_DIM, blk), lambda i: (0, i)),
            pl.BlockSpec(memory_space=pltpu.SMEM),
            pl.BlockSpec(memory_space=pl.ANY),
        ],
        out_specs=[pl.BlockSpec((blk,), lambda i: (i,))] * _DIM
        + [pl.BlockSpec((_DIM, _N_SCORE), lambda i: (0, 0))],
        out_shape=[jax.ShapeDtypeStruct((n,), jnp.float32)] * _DIM
        + [jax.ShapeDtypeStruct((_DIM, _N_SCORE), jnp.float32)],
        scratch_shapes=[
            pltpu.VMEM((_DIM, _N_SCORE), jnp.float32),
            pltpu.VMEM((_DIM, 128), jnp.float32),
            pltpu.SemaphoreType.DMA,
        ],
    )(xt, score_ids, dep)


def _sc_dist_final(user_dims, p_stage, n_stage, user_ids):
    mesh = plsc.VectorSubcoreMesh(core_axis_name="c", subcore_axis_name="s")

    @functools.partial(
        pl.kernel,
        mesh=mesh,
        compiler_params=_SC_PARAMS,
        out_type=[
            jax.ShapeDtypeStruct((_BATCH,), jnp.float32),
            jax.ShapeDtypeStruct((_BATCH,), jnp.float32),
        ],
        scratch_types=[
            pltpu.VMEM((_N_CHUNK, _CHUNK), jnp.int32),
            pltpu.VMEM((_DIM, _B_W), jnp.float32),
            pltpu.VMEM((_DIM, _B_W), jnp.float32),
            pltpu.VMEM((_DIM, _B_W), jnp.float32),
            pltpu.VMEM((_B_W,), jnp.float32),
            pltpu.VMEM((_B_W,), jnp.float32),
            pltpu.SemaphoreType.DMA,
        ],
    )
    def k(*refs):
        user_hbm = refs[:_DIM]
        (p_hbm, n_hbm, uid_hbm, pos_hbm, neg_hbm,
         uid_v, u_v, p_v, n_v, pos_v, neg_v, sem) = refs[_DIM:]
        wid = lax.axis_index("s") * _NC + lax.axis_index("c")
        base = wid * _B_W

        for c in range(_N_CHUNK):
            off = base + c * _CHUNK
            pltpu.sync_copy(uid_hbm.at[pl.ds(off, _CHUNK)], uid_v.at[c])
        pltpu.sync_copy(p_hbm.at[wid], p_v)
        pltpu.sync_copy(n_hbm.at[wid], n_v)

        copies = []
        for d in range(_DIM):
            for c in range(_N_CHUNK):
                dst = pl.ds(c * _CHUNK, _CHUNK)
                copies.append(pltpu.async_copy(
                    user_hbm[d].at[uid_v.at[c]], u_v.at[d].at[dst], sem))
        for cp in copies:
            cp.wait()

        def body(g, carry):
            sl = pl.ds(g * 16, 16)
            accp = jnp.zeros((16,), jnp.float32)
            accn = jnp.zeros((16,), jnp.float32)
            for d in range(_DIM):
                u = u_v[d, sl]
                dp = u - p_v[d, sl]
                dn = u - n_v[d, sl]
                accp = accp + dp * dp
                accn = accn + dn * dn
            pos_v[sl] = accp
            neg_v[sl] = accn
            return carry

        lax.fori_loop(0, _B_W // 16, body, 0, unroll=2)

        pltpu.sync_copy(pos_v, pos_hbm.at[pl.ds(base, _B_W)])
        pltpu.sync_copy(neg_v, neg_hbm.at[pl.ds(base, _B_W)])

    return k(*user_dims, p_stage, n_stage, user_ids)


def _tc_scores(su_t, item_t):
    def body(su_ref, it_ref, out_ref):
        sut = su_ref[...]
        itb = it_ref[...]
        dots = lax.dot_general(sut, itb, (((0,), (0,)), ((), ())),
                               preferred_element_type=jnp.float32)
        su2 = jnp.sum(sut * sut, axis=0)
        it2 = jnp.sum(itb * itb, axis=0)
        out_ref[...] = 2.0 * dots - su2[:, None] - it2[None, :]

    return pl.pallas_call(
        body,
        grid=(pl.cdiv(_NUM_ITEMS, _BI),),
        in_specs=[
            pl.BlockSpec((_DIM, _N_SCORE), lambda i: (0, 0)),
            pl.BlockSpec((_DIM, _BI), lambda i: (0, i)),
        ],
        out_specs=pl.BlockSpec((_N_SCORE, _BI), lambda i: (0, i)),
        out_shape=jax.ShapeDtypeStruct((_N_SCORE, _NUM_ITEMS), jnp.float32),
    )(su_t, item_t)


def kernel(user_embeddings, item_embeddings, user_ids, pos_item_ids,
           neg_item_ids, score_user_ids):
    user_t = user_embeddings.T
    item_t = item_embeddings.T
    item_dims = _tc_split_dims(item_t, 20480)
    p_stage, n_stage = _sc_stage_items(item_dims, pos_item_ids, neg_item_ids)
    outs = _tc_split_user(user_t, 131072, score_user_ids, item_dims[0])
    user_dims, su_t = outs[:_DIM], outs[_DIM]
    pos_d, neg_d = _sc_dist_final(user_dims, p_stage, n_stage, user_ids)
    scores = _tc_scores(su_t, item_t)
    return (pos_d, neg_d, scores)
